# baseline (speedup 1.0000x reference)
import functools

import jax
import jax.numpy as jnp
from jax import lax
from jax.experimental import pallas as pl
from jax.experimental.pallas import tpu as pltpu
from jax.experimental.pallas import tpu_sc as plsc

B, S, D, F, E = 1, 2048, 2048, 2048, 8
T = B * S
DH = D // 2
BT = 64
TP = T + E * BT
NB = TP // BT
BF = 512
NF = F // BF
CR = 128

NC, NS = 2, 16
NW = NC * NS
RPW = TP // NW
CH = 24
CHC = 8
TPW = T // NW


def _router_body(x_ref, wr_ref, ppos_ref, score_ref, gmap_ref, xpk_ref, oh_ref):
    x = x_ref[...]
    lo = lax.bitcast_convert_type(x[:, :DH].astype(jnp.bfloat16), jnp.uint16)
    hi = lax.bitcast_convert_type(x[:, DH:].astype(jnp.bfloat16), jnp.uint16)
    xpk = (hi.astype(jnp.uint32) << 16) | lo.astype(jnp.uint32)
    xpk_ref[...] = lax.bitcast_convert_type(xpk, jnp.int32)
    logits = jnp.dot(x, wr_ref[...], preferred_element_type=jnp.float32)
    top = jnp.argmax(logits, axis=1).astype(jnp.int32)
    score_ref[...] = jax.nn.sigmoid(jnp.max(logits, axis=1))
    oh = (top[:, None] == lax.broadcasted_iota(jnp.int32, (T, E), 1))
    oh = oh.astype(jnp.float32)
    oh_ref[...] = oh
    counts = jnp.sum(oh, axis=0)
    pcounts = jnp.ceil(counts / BT) * BT
    ii = lax.broadcasted_iota(jnp.int32, (E, E), 0)
    jj = lax.broadcasted_iota(jnp.int32, (E, E), 1)
    poff = jnp.sum(jnp.where(jj < ii, pcounts[None, :], 0.0), axis=1)
    ends = poff + pcounts
    bstart = (lax.broadcasted_iota(jnp.int32, (NB, E), 0) * BT).astype(jnp.float32)
    g = jnp.sum((ends[None, :] <= bstart).astype(jnp.int32), axis=1)
    gmap_ref[...] = jnp.minimum(g, E - 1)
    tril = (lax.broadcasted_iota(jnp.int32, (CR, CR), 0)
            >= lax.broadcasted_iota(jnp.int32, (CR, CR), 1)).astype(jnp.float32)

    def chunk(c, carry):
        ohc = oh_ref[pl.ds(c * CR, CR), :]
        rank = jnp.dot(tril, ohc, preferred_element_type=jnp.float32) + carry
        pposc = jnp.sum(ohc * (poff[None, :] + rank - 1.0), axis=1)
        ppos_ref[pl.ds(c * CR, CR)] = pposc.astype(jnp.int32)
        return carry + jnp.sum(ohc, axis=0, keepdims=True)

    lax.fori_loop(0, T // CR, chunk, jnp.zeros((1, E), jnp.float32))


def _router(x, Wr):
    return pl.pallas_call(
        _router_body,
        out_shape=(
            jax.ShapeDtypeStruct((T,), jnp.int32),
            jax.ShapeDtypeStruct((T,), jnp.float32),
            jax.ShapeDtypeStruct((NB,), jnp.int32),
            jax.ShapeDtypeStruct((T, DH), jnp.int32),
        ),
        scratch_shapes=[pltpu.VMEM((T, E), jnp.float32)],
    )(x, Wr)


NBUF = 2
SEG_I = TP // NS
SEG_T = T // NS


CHS = 32


def _dispatch_body(x_hbm, ppos_hbm, score_hbm, xs_hbm, scale_hbm,
                   ppos_v, score_v, zf_v, scale_sh, pidx_v, buf_v,
                   gsems, wsems):
    cid = lax.axis_index("c")
    sid = lax.axis_index("s")
    wid = sid * NC + cid

    tbase = wid * TPW
    pltpu.sync_copy(ppos_hbm.at[pl.ds(tbase, TPW)], pidx_v)
    NCH = TPW // CHS
    g = [None] * NBUF
    w = [None] * NBUF

    def gstart(c, s):
        return pltpu.async_copy(
            x_hbm.at[pl.ds(tbase + c * CHS, CHS)], buf_v.at[s], gsems[s])

    def wstart(c, s):
        return pltpu.async_copy(
            buf_v.at[s], xs_hbm.at[pidx_v.at[pl.ds(c * CHS, CHS)]], wsems[s])

    for c in range(min(NBUF, NCH)):
        g[c] = gstart(c, c)
    for c in range(NCH):
        s = c % NBUF
        g[s].wait()
        w[s] = wstart(c, s)
        nxt = c + NBUF
        if nxt < NCH:
            w[s].wait()
            g[s] = gstart(nxt, s)

    for i in range(SEG_I // 16):
        zf_v[pl.ds(i * 16, 16)] = jnp.zeros((16,), jnp.float32)
    pltpu.sync_copy(zf_v, scale_sh.at[pl.ds(sid * SEG_I, SEG_I)])
    sbase = sid * SEG_T
    pltpu.sync_copy(ppos_hbm.at[pl.ds(sbase, SEG_T)], ppos_v)
    pltpu.sync_copy(score_hbm.at[pl.ds(sbase, SEG_T)], score_v)
    plsc.subcore_barrier()
    pltpu.sync_copy(score_v, scale_sh.at[ppos_v])
    plsc.subcore_barrier()

    @pl.when(jnp.logical_and(sid == 0, cid == 0))
    def _():
        pltpu.sync_copy(scale_sh, scale_hbm)

    for c in range(max(0, NCH - NBUF), NCH):
        w[c % NBUF].wait()


def _dispatch(x, ppos, score):
    mesh = plsc.VectorSubcoreMesh(core_axis_name="c", subcore_axis_name="s")
    return pl.kernel(
        _dispatch_body,
        out_type=(
            jax.ShapeDtypeStruct((TP, DH), jnp.int32),
            jax.ShapeDtypeStruct((TP,), jnp.float32),
        ),
        mesh=mesh,
        scratch_types=[
            pltpu.VMEM((SEG_T,), jnp.int32),
            pltpu.VMEM((SEG_T,), jnp.float32),
            pltpu.VMEM((SEG_I,), jnp.float32),
            pltpu.VMEM_SHARED((TP,), jnp.float32),
            pltpu.VMEM((TPW,), jnp.int32),
            pltpu.VMEM((NBUF, CHS, DH), jnp.int32),
            [pltpu.SemaphoreType.DMA] * NBUF,
            [pltpu.SemaphoreType.DMA] * NBUF,
        ],
    )(x, ppos, score)


def _gmm_body(g_ref, xs_ref, sc_ref, wg_ref, wu_ref, wd_ref, out_ref,
              acc_scr, xs_scr):
    f = pl.program_id(0)
    b = pl.program_id(1)
    sl = pl.ds(b * BT, BT)
    shl = pl.ds(b * BT, BT)

    @pl.when(f == 0)
    def _():
        xs_scr[shl, :] = xs_ref[...]

    xw = xs_scr[shl, :]
    sc = sc_ref[...]
    xlo = lax.bitcast_convert_type(xw << 16, jnp.float32) * sc
    xhi = lax.bitcast_convert_type(xw & jnp.int32(-65536), jnp.float32) * sc
    a = (jnp.dot(xlo, wg_ref[0, :DH, :], preferred_element_type=jnp.float32)
         + jnp.dot(xhi, wg_ref[0, DH:, :], preferred_element_type=jnp.float32))
    u = (jnp.dot(xlo, wu_ref[0, :DH, :], preferred_element_type=jnp.float32)
         + jnp.dot(xhi, wu_ref[0, DH:, :], preferred_element_type=jnp.float32))
    h = a * jax.nn.sigmoid(a) * u
    partial = jnp.dot(h, wd_ref[0], preferred_element_type=jnp.float32)

    @pl.when(f == 0)
    def _():
        acc_scr[sl, :] = partial

    @pl.when(f > 0)
    def _():
        acc_scr[sl, :] += partial

    @pl.when(f == NF - 1)
    def _():
        out_ref[...] = acc_scr[sl, :]


def _gmm(gmap, xs, scale, Wg, Wu, Wd):
    grid_spec = pltpu.PrefetchScalarGridSpec(
        num_scalar_prefetch=1,
        grid=(NF, NB),
        in_specs=[
            pl.BlockSpec((BT, DH), lambda f, b, g: (jnp.where(f == 0, b, 0), 0)),
            pl.BlockSpec((BT, 1), lambda f, b, g: (b, 0)),
            pl.BlockSpec((1, D, BF), lambda f, b, g: (g[b], 0, f)),
            pl.BlockSpec((1, D, BF), lambda f, b, g: (g[b], 0, f)),
            pl.BlockSpec((1, BF, D), lambda f, b, g: (g[b], f, 0)),
        ],
        out_specs=pl.BlockSpec(
            (BT, D), lambda f, b, g: (jnp.where(f == NF - 1, b, 0), 0)),
        scratch_shapes=[
            pltpu.VMEM((TP, D), jnp.float32),
            pltpu.VMEM((TP, DH), jnp.int32),
        ],
    )
    return pl.pallas_call(
        _gmm_body,
        grid_spec=grid_spec,
        out_shape=jax.ShapeDtypeStruct((TP, D), jnp.float32),
        compiler_params=pltpu.CompilerParams(
            vmem_limit_bytes=63 * 1024 * 1024),
    )(gmap, xs, scale, Wg, Wu, Wd)


def _combine_body(outp_hbm, ppos_hbm, out_hbm, idx_v, buf_v, gsems, wsems):
    cid = lax.axis_index("c")
    sid = lax.axis_index("s")
    wid = sid * NC + cid
    base = wid * TPW
    pltpu.sync_copy(ppos_hbm.at[pl.ds(base, TPW)], idx_v)
    NCH = TPW // CHC
    g = [None] * 2
    w = [None] * 2

    def start(c, s):
        g[s] = pltpu.async_copy(
            outp_hbm.at[idx_v.at[pl.ds(c * CHC, CHC)]], buf_v.at[s], gsems[s])

    start(0, 0)
    start(1, 1)
    for c in range(NCH):
        s = c % 2
        g[s].wait()
        w[s] = pltpu.async_copy(
            buf_v.at[s], out_hbm.at[pl.ds(base + c * CHC, CHC)], wsems[s])
        if c + 2 < NCH:
            w[s].wait()
            start(c + 2, s)
    for s in range(2):
        w[s].wait()


def _combine(outp, ppos):
    mesh = plsc.VectorSubcoreMesh(core_axis_name="c", subcore_axis_name="s")
    return pl.kernel(
        _combine_body,
        out_type=jax.ShapeDtypeStruct((T, D), jnp.float32),
        mesh=mesh,
        scratch_types=[
            pltpu.VMEM((TPW,), jnp.int32),
            pltpu.VMEM((2, CHC, D), jnp.float32),
            [pltpu.SemaphoreType.DMA] * 2,
            [pltpu.SemaphoreType.DMA] * 2,
        ],
    )(outp, ppos)


BT2 = 256
M2 = T // BT2


def _shared_body(x_ref, wg_ref, wu_ref, wd_ref, r_ref, out_ref, acc_scr):
    f = pl.program_id(0)
    m = pl.program_id(1)
    xw = x_ref[...]
    xlo = lax.bitcast_convert_type(xw << 16, jnp.float32)
    xhi = lax.bitcast_convert_type(xw & jnp.int32(-65536), jnp.float32)
    a = (jnp.dot(xlo, wg_ref[:DH, :], preferred_element_type=jnp.float32)
         + jnp.dot(xhi, wg_ref[DH:, :], preferred_element_type=jnp.float32))
    u = (jnp.dot(xlo, wu_ref[:DH, :], preferred_element_type=jnp.float32)
         + jnp.dot(xhi, wu_ref[DH:, :], preferred_element_type=jnp.float32))
    h = a * jax.nn.sigmoid(a) * u
    partial = jnp.dot(h, wd_ref[...], preferred_element_type=jnp.float32)
    sl = pl.ds(m * BT2, BT2)

    @pl.when(f == 0)
    def _():
        acc_scr[sl, :] = partial + r_ref[...]

    @pl.when(f > 0)
    def _():
        acc_scr[sl, :] += partial

    @pl.when(f == NF - 1)
    def _():
        out_ref[...] = acc_scr[sl, :]


def _shared(xpk, Wgs, Wus, Wds, routed):
    return pl.pallas_call(
        _shared_body,
        grid=(NF, M2),
        in_specs=[
            pl.BlockSpec((BT2, DH), lambda f, m: (m, 0)),
            pl.BlockSpec((D, BF), lambda f, m: (0, f)),
            pl.BlockSpec((D, BF), lambda f, m: (0, f)),
            pl.BlockSpec((BF, D), lambda f, m: (f, 0)),
            pl.BlockSpec((BT2, D), lambda f, m: (jnp.where(f == 0, m, 0), 0)),
        ],
        out_specs=pl.BlockSpec(
            (BT2, D), lambda f, m: (jnp.where(f == NF - 1, m, 0), 0)),
        out_shape=jax.ShapeDtypeStruct((T, D), jnp.float32),
        scratch_shapes=[pltpu.VMEM((T, D), jnp.float32)],
    )(xpk, Wgs, Wus, Wds, routed)


@jax.jit
def kernel(hidden_states, Wr, Wg, Wu, Wd, Wgs, Wus, Wds):
    x = hidden_states.reshape(T, D)
    ppos, score, gmap, xpk = _router(x, Wr)
    xs32, scale = _dispatch(xpk, ppos, score)
    outp = _gmm(gmap, xs32, scale.reshape(TP, 1), Wg, Wu, Wd)
    routed = _combine(outp, ppos)
    out = _shared(xpk, Wgs, Wus, Wds, routed)
    return out.reshape(B, S, D)


# speedup vs baseline: 1.2525x; 1.2525x over previous
import functools

import jax
import jax.numpy as jnp
from jax import lax
from jax.experimental import pallas as pl
from jax.experimental.pallas import tpu as pltpu
from jax.experimental.pallas import tpu_sc as plsc

B, S, D, F, E = 1, 2048, 2048, 2048, 8
T = B * S
DH = D // 2
BT = 128
TP = T + E * BT
NB = TP // BT
BF = 512
NF = F // BF
CR = 128

NC, NS = 2, 16
NW = NC * NS
RPW = TP // NW
CH = 24
CHC = 8
TPW = T // NW


def _pack_pairs(xlo, xhi):
    lo = lax.bitcast_convert_type(xlo.astype(jnp.bfloat16), jnp.uint16)
    hi = lax.bitcast_convert_type(xhi.astype(jnp.bfloat16), jnp.uint16)
    return lax.bitcast_convert_type(
        (hi.astype(jnp.uint32) << 16) | lo.astype(jnp.uint32), jnp.int32)


def _router_body(x_ref, wr_ref, ppos_ref, gmap_ref, xpk_ref, xspk_ref, oh_ref):
    x = x_ref[...]
    xpk_ref[...] = _pack_pairs(x[:, :DH], x[:, DH:])
    logits = jnp.dot(x, wr_ref[...], preferred_element_type=jnp.float32)
    top = jnp.argmax(logits, axis=1).astype(jnp.int32)
    score = jax.nn.sigmoid(jnp.max(logits, axis=1))
    xr = x * score[:, None]
    xspk_ref[...] = _pack_pairs(xr[:, :DH], xr[:, DH:])
    oh = (top[:, None] == lax.broadcasted_iota(jnp.int32, (T, E), 1))
    oh = oh.astype(jnp.float32)
    oh_ref[...] = oh
    counts = jnp.sum(oh, axis=0)
    pcounts = jnp.ceil(counts / BT) * BT
    ii = lax.broadcasted_iota(jnp.int32, (E, E), 0)
    jj = lax.broadcasted_iota(jnp.int32, (E, E), 1)
    poff = jnp.sum(jnp.where(jj < ii, pcounts[None, :], 0.0), axis=1)
    ends = poff + pcounts
    bstart = (lax.broadcasted_iota(jnp.int32, (NB, E), 0) * BT).astype(jnp.float32)
    g = jnp.sum((ends[None, :] <= bstart).astype(jnp.int32), axis=1)
    gmap_ref[...] = jnp.minimum(g, E - 1)
    tril = (lax.broadcasted_iota(jnp.int32, (CR, CR), 0)
            >= lax.broadcasted_iota(jnp.int32, (CR, CR), 1)).astype(jnp.float32)

    def chunk(c, carry):
        ohc = oh_ref[pl.ds(c * CR, CR), :]
        rank = jnp.dot(tril, ohc, preferred_element_type=jnp.float32) + carry
        pposc = jnp.sum(ohc * (poff[None, :] + rank - 1.0), axis=1)
        ppos_ref[pl.ds(c * CR, CR)] = pposc.astype(jnp.int32)
        return carry + jnp.sum(ohc, axis=0, keepdims=True)

    lax.fori_loop(0, T // CR, chunk, jnp.zeros((1, E), jnp.float32))


def _router(x, Wr):
    return pl.pallas_call(
        _router_body,
        out_shape=(
            jax.ShapeDtypeStruct((T,), jnp.int32),
            jax.ShapeDtypeStruct((NB,), jnp.int32),
            jax.ShapeDtypeStruct((T, DH), jnp.int32),
            jax.ShapeDtypeStruct((T, DH), jnp.int32),
        ),
        scratch_shapes=[pltpu.VMEM((T, E), jnp.float32)],
    )(x, Wr)


NBUF = 2
SEG_I = TP // NS
SEG_T = T // NS


CHS = 32


def _dispatch_body(x_hbm, ppos_hbm, xs_hbm, pidx_v, buf_v, gsems, wsems):
    cid = lax.axis_index("c")
    sid = lax.axis_index("s")
    wid = sid * NC + cid
    tbase = wid * TPW
    pltpu.sync_copy(ppos_hbm.at[pl.ds(tbase, TPW)], pidx_v)
    NCH = TPW // CHS
    g = [None] * NBUF
    w = [None] * NBUF

    def gstart(c, s):
        return pltpu.async_copy(
            x_hbm.at[pl.ds(tbase + c * CHS, CHS)], buf_v.at[s], gsems[s])

    def wstart(c, s):
        return pltpu.async_copy(
            buf_v.at[s], xs_hbm.at[pidx_v.at[pl.ds(c * CHS, CHS)]], wsems[s])

    for c in range(min(NBUF, NCH)):
        g[c] = gstart(c, c)
    for c in range(NCH):
        s = c % NBUF
        g[s].wait()
        w[s] = wstart(c, s)
        nxt = c + NBUF
        if nxt < NCH:
            w[s].wait()
            g[s] = gstart(nxt, s)
    for c in range(max(0, NCH - NBUF), NCH):
        w[c % NBUF].wait()


def _dispatch(xspk, ppos):
    mesh = plsc.VectorSubcoreMesh(core_axis_name="c", subcore_axis_name="s")
    return pl.kernel(
        _dispatch_body,
        out_type=jax.ShapeDtypeStruct((TP, DH), jnp.int32),
        mesh=mesh,
        scratch_types=[
            pltpu.VMEM((TPW,), jnp.int32),
            pltpu.VMEM((NBUF, CHS, DH), jnp.int32),
            [pltpu.SemaphoreType.DMA] * NBUF,
            [pltpu.SemaphoreType.DMA] * NBUF,
        ],
    )(xspk, ppos)


def _unpack_pairs(xw):
    xlo = lax.bitcast_convert_type(xw << 16, jnp.float32)
    xhi = lax.bitcast_convert_type(xw & jnp.int32(-65536), jnp.float32)
    return xlo, xhi


def _gmm_body(g_ref, xs_ref, wg_ref, wu_ref, wd_ref, out_ref,
              acc_scr, xs_scr):
    f = pl.program_id(0)
    b = pl.program_id(1)
    sl = pl.ds(b * BT, BT)

    @pl.when(f == 0)
    def _():
        xs_scr[sl, :] = xs_ref[...]

    xlo, xhi = _unpack_pairs(xs_scr[sl, :])
    a = (jnp.dot(xlo, wg_ref[0, :DH, :], preferred_element_type=jnp.float32)
         + jnp.dot(xhi, wg_ref[0, DH:, :], preferred_element_type=jnp.float32))
    u = (jnp.dot(xlo, wu_ref[0, :DH, :], preferred_element_type=jnp.float32)
         + jnp.dot(xhi, wu_ref[0, DH:, :], preferred_element_type=jnp.float32))
    h = a * jax.nn.sigmoid(a) * u
    partial = jnp.dot(h, wd_ref[0], preferred_element_type=jnp.float32)
    plo, phi = partial[:, :DH], partial[:, DH:]

    @pl.when(f == 0)
    def _():
        acc_scr[sl, :] = _pack_pairs(plo, phi)

    @pl.when(jnp.logical_and(f > 0, f < NF - 1))
    def _():
        alo, ahi = _unpack_pairs(acc_scr[sl, :])
        acc_scr[sl, :] = _pack_pairs(alo + plo, ahi + phi)

    @pl.when(f == NF - 1)
    def _():
        alo, ahi = _unpack_pairs(acc_scr[sl, :])
        out_ref[:, :DH] = alo + plo
        out_ref[:, DH:] = ahi + phi


def _gmm(gmap, xs, Wg, Wu, Wd):
    grid_spec = pltpu.PrefetchScalarGridSpec(
        num_scalar_prefetch=1,
        grid=(NF, NB),
        in_specs=[
            pl.BlockSpec((BT, DH), lambda f, b, g: (jnp.where(f == 0, b, 0), 0)),
            pl.BlockSpec((1, D, BF), lambda f, b, g: (g[b], 0, f)),
            pl.BlockSpec((1, D, BF), lambda f, b, g: (g[b], 0, f)),
            pl.BlockSpec((1, BF, D), lambda f, b, g: (g[b], f, 0)),
        ],
        out_specs=pl.BlockSpec(
            (BT, D), lambda f, b, g: (jnp.where(f == NF - 1, b, 0), 0)),
        scratch_shapes=[
            pltpu.VMEM((TP, DH), jnp.int32),
            pltpu.VMEM((TP, DH), jnp.int32),
        ],
    )
    return pl.pallas_call(
        _gmm_body,
        grid_spec=grid_spec,
        out_shape=jax.ShapeDtypeStruct((TP, D), jnp.float32),
        compiler_params=pltpu.CompilerParams(
            vmem_limit_bytes=63 * 1024 * 1024),
    )(gmap, xs, Wg, Wu, Wd)


def _combine_body(outp_hbm, ppos_hbm, out_hbm, idx_v, buf_v, gsems, wsems):
    cid = lax.axis_index("c")
    sid = lax.axis_index("s")
    wid = sid * NC + cid
    base = wid * TPW
    pltpu.sync_copy(ppos_hbm.at[pl.ds(base, TPW)], idx_v)
    NCH = TPW // CHC
    g = [None] * 2
    w = [None] * 2

    def start(c, s):
        g[s] = pltpu.async_copy(
            outp_hbm.at[idx_v.at[pl.ds(c * CHC, CHC)]], buf_v.at[s], gsems[s])

    start(0, 0)
    start(1, 1)
    for c in range(NCH):
        s = c % 2
        g[s].wait()
        w[s] = pltpu.async_copy(
            buf_v.at[s], out_hbm.at[pl.ds(base + c * CHC, CHC)], wsems[s])
        if c + 2 < NCH:
            w[s].wait()
            start(c + 2, s)
    for s in range(2):
        w[s].wait()


def _combine(outp, ppos):
    mesh = plsc.VectorSubcoreMesh(core_axis_name="c", subcore_axis_name="s")
    return pl.kernel(
        _combine_body,
        out_type=jax.ShapeDtypeStruct((T, D), jnp.float32),
        mesh=mesh,
        scratch_types=[
            pltpu.VMEM((TPW,), jnp.int32),
            pltpu.VMEM((2, CHC, D), jnp.float32),
            [pltpu.SemaphoreType.DMA] * 2,
            [pltpu.SemaphoreType.DMA] * 2,
        ],
    )(outp, ppos)


BT2 = 256
M2 = T // BT2


def _shared_body(x_ref, wg_ref, wu_ref, wd_ref, r_ref, out_ref, acc_scr):
    f = pl.program_id(0)
    m = pl.program_id(1)
    xw = x_ref[...]
    xlo = lax.bitcast_convert_type(xw << 16, jnp.float32)
    xhi = lax.bitcast_convert_type(xw & jnp.int32(-65536), jnp.float32)
    a = (jnp.dot(xlo, wg_ref[:DH, :], preferred_element_type=jnp.float32)
         + jnp.dot(xhi, wg_ref[DH:, :], preferred_element_type=jnp.float32))
    u = (jnp.dot(xlo, wu_ref[:DH, :], preferred_element_type=jnp.float32)
         + jnp.dot(xhi, wu_ref[DH:, :], preferred_element_type=jnp.float32))
    h = a * jax.nn.sigmoid(a) * u
    partial = jnp.dot(h, wd_ref[...], preferred_element_type=jnp.float32)
    sl = pl.ds(m * BT2, BT2)

    @pl.when(f == 0)
    def _():
        acc_scr[sl, :] = partial + r_ref[...]

    @pl.when(f > 0)
    def _():
        acc_scr[sl, :] += partial

    @pl.when(f == NF - 1)
    def _():
        out_ref[...] = acc_scr[sl, :]


def _shared(xpk, Wgs, Wus, Wds, routed):
    return pl.pallas_call(
        _shared_body,
        grid=(NF, M2),
        in_specs=[
            pl.BlockSpec((BT2, DH), lambda f, m: (m, 0)),
            pl.BlockSpec((D, BF), lambda f, m: (0, f)),
            pl.BlockSpec((D, BF), lambda f, m: (0, f)),
            pl.BlockSpec((BF, D), lambda f, m: (f, 0)),
            pl.BlockSpec((BT2, D), lambda f, m: (jnp.where(f == 0, m, 0), 0)),
        ],
        out_specs=pl.BlockSpec(
            (BT2, D), lambda f, m: (jnp.where(f == NF - 1, m, 0), 0)),
        out_shape=jax.ShapeDtypeStruct((T, D), jnp.float32),
        scratch_shapes=[pltpu.VMEM((T, D), jnp.float32)],
    )(xpk, Wgs, Wus, Wds, routed)


@jax.jit
def kernel(hidden_states, Wr, Wg, Wu, Wd, Wgs, Wus, Wds):
    x = hidden_states.reshape(T, D)
    ppos, gmap, xpk, xspk = _router(x, Wr)
    xs32 = _dispatch(xspk, ppos)
    outp = _gmm(gmap, xs32, Wg, Wu, Wd)
    routed = _combine(outp, ppos)
    out = _shared(xpk, Wgs, Wus, Wds, routed)
    return out.reshape(B, S, D)
